# baseline (speedup 1.0000x reference)
import functools

import jax
import jax.numpy as jnp
from jax import lax
from jax.experimental import pallas as pl
from jax.experimental.pallas import tpu as pltpu
from jax.experimental.pallas import tpu_sc as plsc

V, F = 8192, 1024
N = 4096

NC, NS = 2, 16
NW = NC * NS
B_PER_W = N // NW
CHUNK = 8
NCHUNK = B_PER_W // CHUNK
NBUF = 15
LOOKAHEAD = 15

_mesh = plsc.VectorSubcoreMesh(core_axis_name="c", subcore_axis_name="s")


@functools.partial(
    pl.kernel,
    out_type=jax.ShapeDtypeStruct((N, F), jnp.float32),
    mesh=_mesh,
    scratch_types=[
        pltpu.VMEM((B_PER_W,), jnp.int32),
        pltpu.VMEM((NBUF, CHUNK, F), jnp.float32),
        pltpu.SemaphoreType.DMA,
        pltpu.SemaphoreType.DMA,
    ],
)
def _gather_rows(data_hbm, idx_hbm, out_hbm, idx_v, buf_v, gsem, ssem):
    wid = lax.axis_index("s") * NC + lax.axis_index("c")
    base = wid * B_PER_W
    pltpu.sync_copy(idx_hbm.at[pl.ds(base, B_PER_W)], idx_v)

    def gather(c):
        return pltpu.async_copy(
            data_hbm.at[idx_v.at[pl.ds(c * CHUNK, CHUNK)]],
            buf_v.at[c % NBUF], gsem)

    gathers = [None] * NCHUNK
    stores = [None] * NCHUNK
    for c in range(min(LOOKAHEAD, NCHUNK)):
        gathers[c] = gather(c)
    for c in range(NCHUNK):
        gathers[c].wait()
        stores[c] = pltpu.async_copy(
            buf_v.at[c % NBUF], out_hbm.at[pl.ds(base + c * CHUNK, CHUNK)],
            ssem)
        nxt = c + LOOKAHEAD
        if nxt < NCHUNK:
            old = nxt - NBUF
            if old >= 0:
                stores[old].wait()
            gathers[nxt] = gather(nxt)
    for c in range(max(0, NCHUNK - NBUF), NCHUNK):
        stores[c].wait()


def kernel(data, idx):
    return _gather_rows(data, idx[:, 0])
